# Initial kernel scaffold; baseline (speedup 1.0000x reference)
import jax
import jax.numpy as jnp
from jax.experimental import pallas as pl


def kernel(x, edge_index, W_l1, W_r1, b1, W_l2, W_r2, b2, g1, be1, a1, g2, be2, a2, W_res, b_res):
    raise NotImplementedError("write your pallas kernel here")


# speedup vs baseline: 3.1438x; 3.1438x over previous
import jax
import jax.numpy as jnp
from jax import lax
from jax.experimental import pallas as pl
from jax.experimental.pallas import tpu as pltpu
from jax.experimental.pallas import tpu_sc as plsc

N = 10000
D = 128
E = 320000
LANES = 128


def _make_sc_aggregate(nc, ns, with_cnt):
    nw = nc * ns
    kch = -(-(-(-E // nw) // LANES) // 8) * 8
    ept = kch * LANES
    kb = 8
    bpt = -(-(N + 1) // (ns * LANES))
    np_rows = ns * LANES * bpt

    out_type = [jax.ShapeDtypeStruct((nc * np_rows, D), jnp.float32)]
    scratch = [
        pltpu.VMEM_SHARED((np_rows, D), jnp.float32),
        pltpu.VMEM((kb, LANES), jnp.int32),
        pltpu.VMEM((kb, LANES), jnp.int32),
        pltpu.VMEM((LANES, D), jnp.float32),
        pltpu.SemaphoreType.DMA,
    ]
    if with_cnt:
        out_type.append(jax.ShapeDtypeStruct((nw * np_rows,), jnp.float32))
        scratch.append(pltpu.VMEM((np_rows,), jnp.float32))

    mesh = plsc.VectorSubcoreMesh(core_axis_name="c", subcore_axis_name="s",
                                  num_cores=nc, num_subcores=ns)

    def body(*refs):
        if with_cnt:
            (src_hbm, dst_hbm, x_hbm, zx_hbm, zcnt_hbm,
             part_hbm, cnt_hbm, accum, srcv, dstv, rows, sem, cntv) = refs
        else:
            (src_hbm, dst_hbm, x_hbm, zx_hbm,
             part_hbm, accum, srcv, dstv, rows, sem) = refs
        c = lax.axis_index("c")
        s = lax.axis_index("s")
        w = c * ns + s

        pltpu.sync_copy(zx_hbm, rows)
        if with_cnt:
            pltpu.sync_copy(zcnt_hbm, cntv)
        for b in range(bpt):
            r0 = (s * bpt + b) * LANES
            pltpu.sync_copy(rows, accum.at[pl.ds(r0, LANES)])
        plsc.subcore_barrier()

        ones16 = jnp.ones((16,), jnp.float32)

        def group(gi, carry):
            k0 = w * kch + gi * kb
            pltpu.sync_copy(src_hbm.at[pl.ds(k0, kb)], srcv)
            pltpu.sync_copy(dst_hbm.at[pl.ds(k0, kb)], dstv)
            for j in range(kb):
                pltpu.async_copy(x_hbm.at[srcv.at[j]], rows, sem).wait()
                pltpu.sync_copy(rows, accum.at[dstv.at[j]], add=True)
                if with_cnt:
                    for t in range(LANES // 16):
                        idx16 = dstv[j, pl.ds(16 * t, 16)]
                        plsc.addupdate_scatter(cntv, [idx16], ones16)
            return carry

        lax.fori_loop(0, kch // kb, group, 0)
        plsc.subcore_barrier()

        for b in range(bpt):
            r0 = (s * bpt + b) * LANES
            pltpu.sync_copy(accum.at[pl.ds(r0, LANES)], rows)
            pltpu.sync_copy(rows, part_hbm.at[pl.ds(c * np_rows + r0, LANES)])
        if with_cnt:
            pltpu.sync_copy(cntv, cnt_hbm.at[pl.ds(w * np_rows, np_rows)])

    kern = pl.kernel(
        body, out_type=tuple(out_type), mesh=mesh, scratch_types=tuple(scratch),
        compiler_params=pltpu.CompilerParams(needs_layout_passes=False))
    return kern, ept, np_rows


def _tc_layer1(np_rows, nw):
    def body(part, cntp, x, wl, wr, b, g, be, a, o):
        p = part[...]
        psum = p[:N] + p[np_rows:np_rows + N]
        cnt = jnp.sum(cntp[...], axis=0)[:N].reshape(N, 1)
        aggr = psum / jnp.maximum(cnt, 1.0)
        dn = (((1,), (1,)), ((), ()))
        h = (lax.dot_general(aggr, wl[...], dn, preferred_element_type=jnp.float32)
             + lax.dot_general(x[...], wr[...], dn, preferred_element_type=jnp.float32)
             + b[...])
        mean = jnp.mean(h, axis=0, keepdims=True)
        sub = h - a[...] * mean
        var = jnp.mean(sub * sub, axis=0, keepdims=True)
        o[...] = jnp.maximum(g[...] * sub / jnp.sqrt(var + 1e-5) + be[...], 0.0)

    return pl.pallas_call(
        body, out_shape=jax.ShapeDtypeStruct((N, D), jnp.float32))


def _tc_layer2(np_rows, nw):
    def body(part, cntp, x1, x, wl, wr, b, g, be, a, wres, bres, o):
        p = part[...]
        psum = p[:N] + p[np_rows:np_rows + N]
        cnt = jnp.sum(cntp[...], axis=0)[:N].reshape(N, 1)
        aggr = psum / jnp.maximum(cnt, 1.0)
        dn = (((1,), (1,)), ((), ()))
        h = (lax.dot_general(aggr, wl[...], dn, preferred_element_type=jnp.float32)
             + lax.dot_general(x1[...], wr[...], dn, preferred_element_type=jnp.float32)
             + b[...])
        mean = jnp.mean(h, axis=0, keepdims=True)
        sub = h - a[...] * mean
        var = jnp.mean(sub * sub, axis=0, keepdims=True)
        x2 = jnp.maximum(g[...] * sub / jnp.sqrt(var + 1e-5) + be[...], 0.0)
        o[...] = (lax.dot_general(x[...], wres[...], dn,
                                  preferred_element_type=jnp.float32)
                  + bres[...] + x2)

    return pl.pallas_call(
        body, out_shape=jax.ShapeDtypeStruct((N, D), jnp.float32))


def kernel(x, edge_index, W_l1, W_r1, b1, W_l2, W_r2, b2,
           g1, be1, a1, g2, be2, a2, W_res, b_res):
    info = plsc.get_sparse_core_info()
    nc, ns = info.num_cores, info.num_subcores
    nw = nc * ns

    sc1, ept, np_rows = _make_sc_aggregate(nc, ns, with_cnt=True)
    sc2, _, _ = _make_sc_aggregate(nc, ns, with_cnt=False)

    src = edge_index[0].astype(jnp.int32)
    dst = edge_index[1].astype(jnp.int32)
    pad = nw * ept - E
    srcp = jnp.concatenate([src, jnp.zeros((pad,), jnp.int32)])
    dstp = jnp.concatenate([dst, jnp.full((pad,), N, jnp.int32)])
    src2 = srcp.reshape(-1, LANES)
    dst2 = dstp.reshape(-1, LANES)
    zx = jnp.zeros((LANES, D), jnp.float32)
    zcnt = jnp.zeros((np_rows,), jnp.float32)

    b1r, g1r, be1r, a1r = (v.reshape(1, D) for v in (b1, g1, be1, a1))
    b2r, g2r, be2r, a2r = (v.reshape(1, D) for v in (b2, g2, be2, a2))
    bres_r = b_res.reshape(1, D)

    part1, cntf = sc1(src2, dst2, x, zx, zcnt)
    cntp = cntf.reshape(nw, np_rows)
    x1 = _tc_layer1(np_rows, nw)(part1, cntp, x, W_l1, W_r1, b1r, g1r, be1r, a1r)
    (part2,) = sc2(src2, dst2, x1, zx)
    out = _tc_layer2(np_rows, nw)(part2, cntp, x1, x, W_l2, W_r2, b2r,
                                  g2r, be2r, a2r, W_res, bres_r)
    return out


# speedup vs baseline: 3.7821x; 1.2030x over previous
import jax
import jax.numpy as jnp
from jax import lax
from jax.experimental import pallas as pl
from jax.experimental.pallas import tpu as pltpu
from jax.experimental.pallas import tpu_sc as plsc

N = 10000
D = 128
E = 320000
LANES = 128
KB = 8


def _plan(nc, ns):
    nw = nc * ns
    kch = -(-(-(-E // nw) // LANES) // KB) * KB
    np_rows = -(-(N + 1) // (ns * 8)) * (ns * 8)
    rpt = np_rows // ns
    blocks = [LANES] * (rpt // LANES)
    if rpt % LANES:
        blocks.append(rpt % LANES)
    return nw, kch, np_rows, rpt, blocks


def _make_sc_aggregate(nc, ns):
    nw, kch, np_rows, rpt, blocks = _plan(nc, ns)

    mesh = plsc.VectorSubcoreMesh(core_axis_name="c", subcore_axis_name="s",
                                  num_cores=nc, num_subcores=ns)

    def body(src_hbm, dst_hbm, x_hbm, zx_hbm, part_hbm,
             accum, srcv, dstv, rows0, rows1, sem):
        c = lax.axis_index("c")
        s = lax.axis_index("s")
        w = c * ns + s

        pltpu.sync_copy(zx_hbm, rows0)
        r0 = s * rpt
        for blk in blocks:
            pltpu.sync_copy(rows0.at[pl.ds(0, blk)], accum.at[pl.ds(r0, blk)])
            r0 += blk
        plsc.subcore_barrier()

        bufs = (rows0, rows1)

        def group(gi, carry):
            k0 = w * kch + gi * KB
            pltpu.sync_copy(src_hbm.at[pl.ds(k0, KB)], srcv)
            pltpu.sync_copy(dst_hbm.at[pl.ds(k0, KB)], dstv)
            d = pltpu.async_copy(x_hbm.at[srcv.at[0]], bufs[0], sem)
            for j in range(KB):
                d.wait()
                if j + 1 < KB:
                    d = pltpu.async_copy(x_hbm.at[srcv.at[j + 1]],
                                         bufs[(j + 1) % 2], sem)
                pltpu.sync_copy(bufs[j % 2], accum.at[dstv.at[j]], add=True)
            return carry

        lax.fori_loop(0, kch // KB, group, 0)
        plsc.subcore_barrier()

        r0 = s * rpt
        for blk in blocks:
            pltpu.sync_copy(accum.at[pl.ds(r0, blk)], rows0.at[pl.ds(0, blk)])
            pltpu.sync_copy(rows0.at[pl.ds(0, blk)],
                            part_hbm.at[pl.ds(c * np_rows + r0, blk)])
            r0 += blk

    return pl.kernel(
        body,
        out_type=jax.ShapeDtypeStruct((nc * np_rows, D), jnp.float32),
        mesh=mesh,
        scratch_types=(
            pltpu.VMEM_SHARED((np_rows, D), jnp.float32),
            pltpu.VMEM((KB, LANES), jnp.int32),
            pltpu.VMEM((KB, LANES), jnp.int32),
            pltpu.VMEM((LANES, D), jnp.float32),
            pltpu.VMEM((LANES, D), jnp.float32),
            pltpu.SemaphoreType.DMA,
        ),
        compiler_params=pltpu.CompilerParams(needs_layout_passes=False))


def _make_sc_count(nc, ns):
    nw, kch, np_rows, _, _ = _plan(nc, ns)
    mesh = plsc.VectorSubcoreMesh(core_axis_name="c", subcore_axis_name="s",
                                  num_cores=nc, num_subcores=ns)

    def body(dst_hbm, zcnt_hbm, cnt_hbm, dstv, cntv):
        c = lax.axis_index("c")
        s = lax.axis_index("s")
        w = c * ns + s
        pltpu.sync_copy(zcnt_hbm, cntv)
        ones16 = jnp.ones((16,), jnp.float32)

        def group(gi, carry):
            k0 = w * kch + gi * KB
            pltpu.sync_copy(dst_hbm.at[pl.ds(k0, KB)], dstv)
            for j in range(KB):
                for t in range(LANES // 16):
                    idx16 = dstv[j, pl.ds(16 * t, 16)]
                    plsc.addupdate_scatter(cntv, [idx16], ones16)
            return carry

        lax.fori_loop(0, kch // KB, group, 0)
        pltpu.sync_copy(cntv, cnt_hbm.at[pl.ds(w * np_rows, np_rows)])

    return pl.kernel(
        body,
        out_type=jax.ShapeDtypeStruct((nw * np_rows,), jnp.float32),
        mesh=mesh,
        scratch_types=(
            pltpu.VMEM((KB, LANES), jnp.int32),
            pltpu.VMEM((np_rows,), jnp.float32),
        ),
        compiler_params=pltpu.CompilerParams(needs_layout_passes=False))


def _tc_layer1(np_rows):
    def body(part, cntp, x, wl, wr, b, g, be, a, o):
        p = part[...]
        psum = p[:N] + p[np_rows:np_rows + N]
        cnt = jnp.sum(cntp[...], axis=0)[:N].reshape(N, 1)
        aggr = psum / jnp.maximum(cnt, 1.0)
        dn = (((1,), (1,)), ((), ()))
        h = (lax.dot_general(aggr, wl[...], dn, preferred_element_type=jnp.float32)
             + lax.dot_general(x[...], wr[...], dn, preferred_element_type=jnp.float32)
             + b[...])
        mean = jnp.mean(h, axis=0, keepdims=True)
        sub = h - a[...] * mean
        var = jnp.mean(sub * sub, axis=0, keepdims=True)
        o[...] = jnp.maximum(g[...] * sub / jnp.sqrt(var + 1e-5) + be[...], 0.0)

    return pl.pallas_call(
        body, out_shape=jax.ShapeDtypeStruct((N, D), jnp.float32))


def _tc_layer2(np_rows):
    def body(part, cntp, x1, x, wl, wr, b, g, be, a, wres, bres, o):
        p = part[...]
        psum = p[:N] + p[np_rows:np_rows + N]
        cnt = jnp.sum(cntp[...], axis=0)[:N].reshape(N, 1)
        aggr = psum / jnp.maximum(cnt, 1.0)
        dn = (((1,), (1,)), ((), ()))
        h = (lax.dot_general(aggr, wl[...], dn, preferred_element_type=jnp.float32)
             + lax.dot_general(x1[...], wr[...], dn, preferred_element_type=jnp.float32)
             + b[...])
        mean = jnp.mean(h, axis=0, keepdims=True)
        sub = h - a[...] * mean
        var = jnp.mean(sub * sub, axis=0, keepdims=True)
        x2 = jnp.maximum(g[...] * sub / jnp.sqrt(var + 1e-5) + be[...], 0.0)
        o[...] = (lax.dot_general(x[...], wres[...], dn,
                                  preferred_element_type=jnp.float32)
                  + bres[...] + x2)

    return pl.pallas_call(
        body, out_shape=jax.ShapeDtypeStruct((N, D), jnp.float32))


def kernel(x, edge_index, W_l1, W_r1, b1, W_l2, W_r2, b2,
           g1, be1, a1, g2, be2, a2, W_res, b_res):
    info = plsc.get_sparse_core_info()
    nc, ns = info.num_cores, info.num_subcores
    nw, kch, np_rows, _, _ = _plan(nc, ns)

    sc_aggr = _make_sc_aggregate(nc, ns)
    sc_cnt = _make_sc_count(nc, ns)

    src = edge_index[0].astype(jnp.int32)
    dst = edge_index[1].astype(jnp.int32)
    pad = nw * kch * LANES - E
    srcp = jnp.concatenate([src, jnp.zeros((pad,), jnp.int32)])
    dstp = jnp.concatenate([dst, jnp.full((pad,), N, jnp.int32)])
    src2 = srcp.reshape(-1, LANES)
    dst2 = dstp.reshape(-1, LANES)
    zx = jnp.zeros((LANES, D), jnp.float32)
    zcnt = jnp.zeros((np_rows,), jnp.float32)

    b1r, g1r, be1r, a1r = (v.reshape(1, D) for v in (b1, g1, be1, a1))
    b2r, g2r, be2r, a2r = (v.reshape(1, D) for v in (b2, g2, be2, a2))
    bres_r = b_res.reshape(1, D)

    cntf = sc_cnt(dst2, zcnt)
    cntp = cntf.reshape(nw, np_rows)
    part1 = sc_aggr(src2, dst2, x, zx)
    x1 = _tc_layer1(np_rows)(part1, cntp, x, W_l1, W_r1, b1r, g1r, be1r, a1r)
    part2 = sc_aggr(src2, dst2, x1, zx)
    out = _tc_layer2(np_rows)(part2, cntp, x1, x, W_l2, W_r2, b2r,
                              g2r, be2r, a2r, W_res, bres_r)
    return out
